# Initial kernel scaffold; baseline (speedup 1.0000x reference)
import jax
import jax.numpy as jnp
from jax.experimental import pallas as pl


def kernel(x, edge_index, batch, W1, b1, W2, b2, Wl, bl):
    raise NotImplementedError("write your pallas kernel here")


# speedup vs baseline: 33.2403x; 33.2403x over previous
import functools

import jax
import jax.numpy as jnp
from jax import lax
from jax.experimental import pallas as pl
from jax.experimental.pallas import tpu as pltpu
from jax.experimental.pallas import tpu_sc as plsc

_NC = 2
_NS = 16
_K = 128
_CB = 4
_G = 128


def _round_up(a, b):
    return (a + b - 1) // b * b


def _sc_degree(dst3, n_pad):
    ch = dst3.shape[1]
    rpt = n_pad // _NS
    mesh = plsc.VectorSubcoreMesh(core_axis_name="c", subcore_axis_name="s")

    @functools.partial(
        pl.kernel,
        out_type=jax.ShapeDtypeStruct((_NC, n_pad), jnp.float32),
        mesh=mesh,
        scratch_types=[
            pltpu.VMEM((ch, _K), jnp.int32),
            pltpu.VMEM((_K,), jnp.float32),
            pltpu.VMEM((rpt,), jnp.float32),
            pltpu.VMEM_SHARED((n_pad,), jnp.float32),
        ],
    )
    def k(dst_hbm, out_hbm, dst_v, ones_v, zbuf_v, acc):
        c = lax.axis_index("c")
        s = lax.axis_index("s")
        wid = c * _NS + s
        pltpu.sync_copy(dst_hbm.at[wid], dst_v)
        def fill(i, _):
            ones_v[pl.ds(i * 16, 16)] = jnp.ones((16,), jnp.float32)
            zbuf_v[pl.ds(i * 16, 16)] = jnp.zeros((16,), jnp.float32)
            return 0
        lax.fori_loop(0, _K // 16, fill, 0)
        def fillz(i, _):
            zbuf_v[pl.ds(i * 16, 16)] = jnp.zeros((16,), jnp.float32)
            return 0
        lax.fori_loop(_K // 16, rpt // 16, fillz, 0)
        base = s * rpt
        pltpu.sync_copy(zbuf_v, acc.at[pl.ds(base, rpt)])
        plsc.subcore_barrier()
        def step(j, _):
            pltpu.sync_copy(ones_v, acc.at[dst_v.at[j]], add=True)
            return 0
        lax.fori_loop(0, ch, step, 0)
        plsc.subcore_barrier()
        pltpu.sync_copy(acc.at[pl.ds(base, rpt)],
                        out_hbm.at[c, pl.ds(base, rpt)])

    return k(dst3)


def _sc_scatter(table, src4, dst4, zeros_hbm, n_pad):
    nb = src4.shape[1]
    h = table.shape[1]
    rpt = n_pad // _NS
    mesh = plsc.VectorSubcoreMesh(core_axis_name="c", subcore_axis_name="s")

    @functools.partial(
        pl.kernel,
        out_type=jax.ShapeDtypeStruct((_NC, n_pad, h), jnp.float32),
        mesh=mesh,
        scratch_types=[
            pltpu.VMEM((2, _CB, _K), jnp.int32),
            pltpu.VMEM((2, _CB, _K), jnp.int32),
            pltpu.VMEM((2, _K, h), jnp.float32),
            pltpu.VMEM_SHARED((n_pad, h), jnp.float32),
            pltpu.SemaphoreType.DMA,
            pltpu.SemaphoreType.DMA,
            pltpu.SemaphoreType.DMA,
            pltpu.SemaphoreType.DMA,
        ],
    )
    def k(tab_hbm, src_hbm, dst_hbm, z_hbm, out_hbm,
          sidx, didx, rows, acc, semg0, semg1, semi0, semi1):
        c = lax.axis_index("c")
        s = lax.axis_index("s")
        wid = c * _NS + s
        semg = (semg0, semg1)
        semi = (semi0, semi1)

        pltpu.sync_copy(z_hbm, rows.at[0])
        base = s * rpt
        for i in range(rpt // _K):
            pltpu.sync_copy(rows.at[0], acc.at[pl.ds(base + i * _K, _K)])
        pltpu.sync_copy(src_hbm.at[wid, 0], sidx.at[0])
        pltpu.sync_copy(dst_hbm.at[wid, 0], didx.at[0])
        plsc.subcore_barrier()
        pltpu.async_copy(tab_hbm.at[sidx.at[0, 0]], rows.at[0], semg0)

        def outer(bp, _):
            for sub in (0, 1):
                b = 2 * bp + sub
                cur, nxt = sub, 1 - sub

                @pl.when(b < nb - 1)
                def _():
                    pltpu.async_copy(src_hbm.at[wid, b + 1], sidx.at[nxt],
                                     semi[nxt])
                    pltpu.async_copy(dst_hbm.at[wid, b + 1], didx.at[nxt],
                                     semi[nxt])
                for j in range(_CB):
                    buf = j % 2
                    if j < _CB - 1:
                        pltpu.async_copy(tab_hbm.at[sidx.at[cur, j + 1]],
                                         rows.at[1 - buf], semg[1 - buf])
                    else:
                        @pl.when(b < nb - 1)
                        def _():
                            pltpu.make_async_copy(src_hbm.at[wid, b + 1],
                                                  sidx.at[nxt],
                                                  semi[nxt]).wait()
                            pltpu.make_async_copy(dst_hbm.at[wid, b + 1],
                                                  didx.at[nxt],
                                                  semi[nxt]).wait()
                            pltpu.async_copy(tab_hbm.at[sidx.at[nxt, 0]],
                                             rows.at[1 - buf], semg[1 - buf])
                    pltpu.make_async_copy(tab_hbm.at[sidx.at[cur, j]],
                                          rows.at[buf], semg[buf]).wait()
                    pltpu.sync_copy(rows.at[buf], acc.at[didx.at[cur, j]],
                                    add=True)
            return 0
        lax.fori_loop(0, nb // 2, outer, 0)
        plsc.subcore_barrier()
        pltpu.sync_copy(acc.at[pl.ds(base, rpt)],
                        out_hbm.at[c, pl.ds(base, rpt)])

    return k(table, src4, dst4, zeros_hbm)


def _dinv(deg_ref):
    d = deg_ref[...]
    return lax.rsqrt(jnp.sum(d, axis=1, keepdims=True) + 1.0)


def _tc1_body(x_ref, w_ref, deg_ref, o_ref):
    o_ref[...] = jnp.dot(x_ref[...], w_ref[...],
                         preferred_element_type=jnp.float32) * _dinv(deg_ref)


def _tc2_body(p_ref, hs_ref, deg_ref, b_ref, w_ref, o_ref):
    dinv = _dinv(deg_ref)
    agg = p_ref[0] + p_ref[1] + hs_ref[...]
    hcur = jnp.maximum(agg * dinv + b_ref[...], 0.0)
    o_ref[...] = jnp.dot(hcur, w_ref[...],
                         preferred_element_type=jnp.float32) * dinv


def _tc3_body(p_ref, hs_ref, deg_ref, b_ref, batch_ref, wl_ref, bl_ref,
              o_ref, pool_acc, cnt_acc):
    j = pl.program_id(0)
    nsteps = pl.num_programs(0)

    @pl.when(j == 0)
    def _():
        pool_acc[...] = jnp.zeros_like(pool_acc)
        cnt_acc[...] = jnp.zeros_like(cnt_acc)

    dinv = _dinv(deg_ref)
    agg = p_ref[0] + p_ref[1] + hs_ref[...]
    hcur = jnp.maximum(agg * dinv + b_ref[...], 0.0)
    gid = lax.broadcasted_iota(jnp.int32, (_G, hcur.shape[0]), 0)
    m = (gid == batch_ref[0]).astype(jnp.float32)
    pool_acc[...] += lax.dot_general(m, hcur, (((1,), (0,)), ((), ())),
                                     preferred_element_type=jnp.float32)
    cnt_acc[...] += jnp.sum(m, axis=1, keepdims=True)

    @pl.when(j == nsteps - 1)
    def _():
        pooled = pool_acc[...] / jnp.maximum(cnt_acc[...], 1.0)
        o_ref[...] = (jnp.dot(pooled, wl_ref[...],
                              preferred_element_type=jnp.float32)
                      + bl_ref[...])


def kernel(x, edge_index, batch, W1, b1, W2, b2, Wl, bl):
    n, d = x.shape
    h = W1.shape[1]
    o = Wl.shape[1]
    e = edge_index.shape[1]
    src = edge_index[0]
    dst = edge_index[1]

    n_pad = _round_up(n + 1, _NS * _K)
    n_junk = n_pad - n

    workers = _NC * _NS
    blk = _CB * _K
    nb = _round_up(_round_up(e, workers * blk) // (workers * blk), 2)
    e_pad = workers * nb * blk
    npd = e_pad - e
    fill_src = (jnp.arange(npd, dtype=jnp.int32) * 2003) % n
    fill_dst = n + (jnp.arange(npd, dtype=jnp.int32) % n_junk)
    src4 = jnp.concatenate([src, fill_src]).reshape(workers, nb, _CB, _K)
    dst4 = jnp.concatenate([dst, fill_dst]).reshape(workers, nb, _CB, _K)
    dst3 = dst4.reshape(workers, nb * _CB, _K)

    zeros_hbm = jnp.zeros((_K, h), jnp.float32)

    degp = _sc_degree(dst3, n_pad)
    deg2 = degp.T[:n]

    br = 1000
    grid = n // br
    hs1 = pl.pallas_call(
        _tc1_body,
        grid=(grid,),
        in_specs=[
            pl.BlockSpec((br, d), lambda j: (j, 0)),
            pl.BlockSpec((d, h), lambda j: (0, 0)),
            pl.BlockSpec((br, 2), lambda j: (j, 0)),
        ],
        out_specs=pl.BlockSpec((br, h), lambda j: (j, 0)),
        out_shape=jax.ShapeDtypeStruct((n, h), jnp.float32),
    )(x, W1, deg2)

    p1 = _sc_scatter(hs1, src4, dst4, zeros_hbm, n_pad)

    hs2 = pl.pallas_call(
        _tc2_body,
        grid=(grid,),
        in_specs=[
            pl.BlockSpec((_NC, br, h), lambda j: (0, j, 0)),
            pl.BlockSpec((br, h), lambda j: (j, 0)),
            pl.BlockSpec((br, 2), lambda j: (j, 0)),
            pl.BlockSpec((1, h), lambda j: (0, 0)),
            pl.BlockSpec((h, h), lambda j: (0, 0)),
        ],
        out_specs=pl.BlockSpec((br, h), lambda j: (j, 0)),
        out_shape=jax.ShapeDtypeStruct((n, h), jnp.float32),
    )(p1, hs1, deg2, b1.reshape(1, h), W2)

    p2 = _sc_scatter(hs2, src4, dst4, zeros_hbm, n_pad)

    out = pl.pallas_call(
        _tc3_body,
        grid=(grid,),
        in_specs=[
            pl.BlockSpec((_NC, br, h), lambda j: (0, j, 0)),
            pl.BlockSpec((br, h), lambda j: (j, 0)),
            pl.BlockSpec((br, 2), lambda j: (j, 0)),
            pl.BlockSpec((1, h), lambda j: (0, 0)),
            pl.BlockSpec((1, 1, br), lambda j: (j, 0, 0)),
            pl.BlockSpec((h, o), lambda j: (0, 0)),
            pl.BlockSpec((1, o), lambda j: (0, 0)),
        ],
        out_specs=pl.BlockSpec((_G, o), lambda j: (0, 0)),
        out_shape=jax.ShapeDtypeStruct((_G, o), jnp.float32),
        scratch_shapes=[
            pltpu.VMEM((_G, h), jnp.float32),
            pltpu.VMEM((_G, 1), jnp.float32),
        ],
    )(p2, hs2, deg2, b2.reshape(1, h), batch.reshape(grid, 1, br), Wl,
      bl.reshape(1, o))

    return out
